# baseline (speedup 1.0000x reference)
import functools

import jax
import jax.numpy as jnp
from jax import lax
import jax.experimental.pallas as pl
from jax.experimental.pallas import tpu as pltpu
from jax.experimental.pallas import tpu_sc as plsc

_NC = 2
_NS = 16
_W = 80
_G = 64
_RB = 2000


_NB = 4
_PW = 16


def _sc_degree(dst_r, ones_n, ones_w):
    np8 = ones_n.shape[0]
    nw = dst_r.shape[1]
    nblk = nw // _NB
    mesh = plsc.VectorSubcoreMesh(core_axis_name="c", subcore_axis_name="s")

    @functools.partial(
        pl.kernel,
        out_type=jax.ShapeDtypeStruct((np8,), jnp.float32),
        mesh=mesh,
        scratch_types=[
            pltpu.VMEM((nw, _W), jnp.int32),
            pltpu.VMEM((_W,), jnp.float32),
            pltpu.VMEM_SHARED((np8,), jnp.float32),
            [pltpu.SemaphoreType.DMA] * _NB,
        ],
    )
    def k(dst_h, ones_n_h, ones_w_h, out_h, dst2, onev, acc, sems):
        cid = lax.axis_index("c")
        sid = lax.axis_index("s")

        @pl.when(cid == 0)
        def _():
            @pl.when(sid == 0)
            def _():
                pltpu.sync_copy(ones_n_h, acc)
            pltpu.sync_copy(ones_w_h, onev)
            pltpu.sync_copy(dst_h.at[sid], dst2)
            plsc.subcore_barrier()

            def outer(k_):
                base = k_ * _NB
                for b in range(_NB):
                    @pl.when(k_ > 0)
                    def _():
                        pltpu.make_async_copy(
                            onev, acc.at[dst2.at[base - _NB + b]],
                            sems[b]).wait()
                    pltpu.async_copy(onev, acc.at[dst2.at[base + b]],
                                     sems[b], add=True)

            pl.loop(0, nblk)(outer)
            for b in range(_NB):
                pltpu.make_async_copy(
                    onev, acc.at[dst2.at[(nblk - 1) * _NB + b]],
                    sems[b]).wait()
            plsc.subcore_barrier()

            @pl.when(sid == 0)
            def _():
                pltpu.sync_copy(acc, out_h)

    return k(dst_r, ones_n, ones_w)


def _sc_aggregate(src, dst, g_flat, zeros_rows, *, n, ch, n_chunks):
    split = n_chunks == 1
    cpc = 1 if split else n_chunks // _NC
    nw = src.shape[1]
    n_out = _NC * n if split else n_chunks * n
    rpt_a = -(-(n // _NS) // 8) * 8
    rpt_b = n - (_NS - 1) * rpt_a
    mesh = plsc.VectorSubcoreMesh(core_axis_name="c", subcore_axis_name="s")

    npass = nw // _PW

    @functools.partial(
        pl.kernel,
        out_type=jax.ShapeDtypeStruct((n_out, ch), jnp.float32),
        mesh=mesh,
        scratch_types=[
            pltpu.VMEM((2 * _PW, _W), jnp.int32),
            pltpu.VMEM((2 * _PW, _W), jnp.int32),
            pltpu.VMEM((_NB, _W), jnp.int32),
            pltpu.VMEM((_NB * _W, ch), jnp.float32),
            pltpu.VMEM_SHARED((n + 8, ch), jnp.float32),
            [pltpu.SemaphoreType.DMA] * _NB,
            [pltpu.SemaphoreType.DMA] * 2,
        ],
    )
    def k(src_h, dst_h, g_h, z_h, out_h, src2, dst2, idxg_a, rows_a, acc,
          sems, psems):
        rows = [rows_a.at[pl.ds(b * _W, _W)] for b in range(_NB)]
        idxg = [idxg_a.at[b] for b in range(_NB)]
        cid = lax.axis_index("c")
        sid = lax.axis_index("s")
        slab = cid * _NS + sid if split else sid
        rbase = sid * rpt_a
        last = sid == _NS - 1

        for ci in range(cpc):
            chunk = cid * cpc + ci
            coff_in = 0 if split else chunk * n
            coff = cid * n if split else chunk * n

            def issue_gather(b, w):
                for j in range(_W // 16):
                    sl = pl.ds(j * 16, 16)
                    idxg[b][sl] = (src2[w, sl] if split
                                   else src2[w, sl] + coff_in)
                pltpu.async_copy(g_h.at[idxg[b]], rows[b], sems[b])

            @pl.when(jnp.logical_not(last))
            def _():
                pltpu.sync_copy(z_h, acc.at[pl.ds(rbase, rpt_a)])

            @pl.when(last)
            def _():
                pltpu.sync_copy(z_h.at[pl.ds(0, rpt_b)],
                                acc.at[pl.ds(rbase, rpt_b)])

            plsc.subcore_barrier()

            pltpu.sync_copy(src_h.at[slab, pl.ds(0, _PW)],
                            src2.at[pl.ds(0, _PW)])
            pltpu.sync_copy(dst_h.at[slab, pl.ds(0, _PW)],
                            dst2.at[pl.ds(0, _PW)])

            for p_ in range(npass):
                boff = (p_ % 2) * _PW
                noff = ((p_ + 1) % 2) * _PW
                if p_ + 1 < npass:
                    pltpu.async_copy(
                        src_h.at[slab, pl.ds((p_ + 1) * _PW, _PW)],
                        src2.at[pl.ds(noff, _PW)], psems[0])
                    pltpu.async_copy(
                        dst_h.at[slab, pl.ds((p_ + 1) * _PW, _PW)],
                        dst2.at[pl.ds(noff, _PW)], psems[1])

                for b in range(_NB):
                    issue_gather(b, boff + b)

                def outer(k_):
                    base = k_ * _NB
                    for b in range(_NB):
                        pltpu.make_async_copy(g_h.at[idxg[b]], rows[b],
                                              sems[b]).wait()
                        pltpu.async_copy(rows[b],
                                         acc.at[dst2.at[boff + base + b]],
                                         sems[b], add=True)
                    for b in range(_NB):
                        pltpu.make_async_copy(rows[b],
                                              acc.at[dst2.at[boff + base + b]],
                                              sems[b]).wait()

                        @pl.when(base + _NB + b < _PW)
                        def _():
                            issue_gather(b, boff + base + _NB + b)

                pl.loop(0, _PW // _NB)(outer)

                if p_ + 1 < npass:
                    pltpu.make_async_copy(
                        src_h.at[slab, pl.ds((p_ + 1) * _PW, _PW)],
                        src2.at[pl.ds(noff, _PW)], psems[0]).wait()
                    pltpu.make_async_copy(
                        dst_h.at[slab, pl.ds((p_ + 1) * _PW, _PW)],
                        dst2.at[pl.ds(noff, _PW)], psems[1]).wait()

            plsc.subcore_barrier()

            @pl.when(jnp.logical_not(last))
            def _():
                pltpu.sync_copy(acc.at[pl.ds(rbase, rpt_a)],
                                out_h.at[pl.ds(coff + rbase, rpt_a)])

            @pl.when(last)
            def _():
                pltpu.sync_copy(acc.at[pl.ds(rbase, rpt_b)],
                                out_h.at[pl.ds(coff + rbase, rpt_b)])

    return k(src, dst, g_flat, zeros_rows)


def _tc_gmm(p, wc, *, deg=None):
    n, din = p.shape
    n_chunks, _, ch = wc.shape

    def body_plain(p_ref, w_ref, o_ref):
        o_ref[0] = jnp.dot(p_ref[...], w_ref[0],
                           preferred_element_type=jnp.float32)

    def body_scaled(p_ref, w_ref, d_ref, o_ref):
        pv = p_ref[...] * lax.rsqrt(d_ref[...])
        o_ref[0] = jnp.dot(pv, w_ref[0], preferred_element_type=jnp.float32)

    in_specs = [
        pl.BlockSpec((_RB, din), lambda i, c: (i, 0)),
        pl.BlockSpec((1, din, ch), lambda i, c: (c, 0, 0)),
    ]
    args = [p, wc]
    body = body_plain
    if deg is not None:
        in_specs.append(pl.BlockSpec((_RB, 1), lambda i, c: (i, 0)))
        args.append(deg)
        body = body_scaled
    return pl.pallas_call(
        body,
        grid=(n // _RB, n_chunks),
        in_specs=in_specs,
        out_specs=pl.BlockSpec((1, _RB, ch), lambda i, c: (c, i, 0)),
        out_shape=jax.ShapeDtypeStruct((n_chunks, n, ch), jnp.float32),
    )(*args)


def _epi(a_ref, g_ref, d_ref, b_ref):
    dinv = lax.rsqrt(d_ref[...])
    a = a_ref[...]
    gg = g_ref[...]
    nc = gg.shape[0]
    if a.shape[0] == nc:
        parts = [a[i] + gg[i] for i in range(nc)]
    else:
        parts = [a[0] + a[1] + gg[0]]
    sc = parts[0] if len(parts) == 1 else jnp.concatenate(parts, axis=1)
    return jnp.maximum(dinv * sc + b_ref[...], 0.0), dinv


def _tc_gcn_fused(agg, g, deg, b, wc):
    nc_in, n, ch = g.shape
    m = agg.shape[0]
    nc_out, din, cho = wc.shape

    def body(a_ref, g_ref, d_ref, b_ref, w_ref, o_ref):
        h, dinv = _epi(a_ref, g_ref, d_ref, b_ref)
        p = h * dinv
        for c in range(nc_out):
            o_ref[c] = jnp.dot(p, w_ref[c], preferred_element_type=jnp.float32)

    return pl.pallas_call(
        body,
        grid=(n // _RB,),
        in_specs=[
            pl.BlockSpec((m, _RB, ch), lambda i: (0, i, 0)),
            pl.BlockSpec((nc_in, _RB, ch), lambda i: (0, i, 0)),
            pl.BlockSpec((_RB, 1), lambda i: (i, 0)),
            pl.BlockSpec((1, din), lambda i: (0, 0)),
            pl.BlockSpec((nc_out, din, cho), lambda i: (0, 0, 0)),
        ],
        out_specs=pl.BlockSpec((nc_out, _RB, cho), lambda i: (0, i, 0)),
        out_shape=jax.ShapeDtypeStruct((nc_out, n, cho), jnp.float32),
    )(agg, g, deg, b, wc)


def _tc_pool_fused(agg, g, deg, b, batch2):
    nc, n, ch = g.shape
    dout = nc * ch

    def body(a_ref, g_ref, d_ref, b_ref, bt_ref, o_ref):
        @pl.when(pl.program_id(0) == 0)
        def _():
            o_ref[...] = jnp.full((_G, dout), -jnp.inf, jnp.float32)

        hv, _ = _epi(a_ref, g_ref, d_ref, b_ref)
        bv = bt_ref[...]
        lo = bv[0, 0]
        hi = bv[_RB - 1, 0]

        def gbody(gi, _):
            mk = bv == gi
            part = jnp.max(jnp.where(mk, hv, -jnp.inf), axis=0, keepdims=True)
            cur = o_ref[pl.ds(gi, 1), :]
            o_ref[pl.ds(gi, 1), :] = jnp.maximum(cur, part)
            return ()

        lax.fori_loop(lo, hi + 1, gbody, ())

    return pl.pallas_call(
        body,
        grid=(n // _RB,),
        in_specs=[
            pl.BlockSpec((nc, _RB, ch), lambda i: (0, i, 0)),
            pl.BlockSpec((nc, _RB, ch), lambda i: (0, i, 0)),
            pl.BlockSpec((_RB, 1), lambda i: (i, 0)),
            pl.BlockSpec((1, dout), lambda i: (0, 0)),
            pl.BlockSpec((_RB, 1), lambda i: (i, 0)),
        ],
        out_specs=pl.BlockSpec((_G, dout), lambda i: (0, 0)),
        out_shape=jax.ShapeDtypeStruct((_G, dout), jnp.float32),
    )(agg, g, deg, b, batch2)


def _tc_mlp(pooled, wf1, bf1, wf2, bf2):
    def body(p_ref, w1_ref, b1_ref, w2_ref, b2_ref, o_ref):
        t = jnp.dot(p_ref[...], w1_ref[...], preferred_element_type=jnp.float32)
        t = jnp.maximum(t + b1_ref[...], 0.0)
        o_ref[...] = jnp.dot(t, w2_ref[...],
                             preferred_element_type=jnp.float32) + b2_ref[...]

    g, dout = pooled.shape[0], wf2.shape[1]
    return pl.pallas_call(
        body,
        out_shape=jax.ShapeDtypeStruct((g, dout), jnp.float32),
    )(pooled, wf1, bf1.reshape(1, -1), wf2, bf2.reshape(1, -1))


def kernel(x, edge_index, batch, W1, b1, W2, b2, W3, b3, Wf1, bf1, Wf2, bf2):
    n = x.shape[0]
    e = edge_index.shape[1]
    src = edge_index[0].astype(jnp.int32)
    dst = edge_index[1].astype(jnp.int32)

    ept16 = -(-(e // _NS) // (_W * _PW)) * (_W * _PW)
    padn = _NS * ept16 - e
    pad_src = (jnp.arange(padn, dtype=jnp.int32) % n)
    pad_dst = n + (jnp.arange(padn, dtype=jnp.int32) % 8)

    def windows(a, pad, groups):
        return jnp.concatenate(
            [a.reshape(groups, -1), pad.reshape(groups, -1)], axis=1
        ).reshape(groups, -1, _W)

    src16 = windows(src, pad_src, _NS)
    dst16 = windows(dst, pad_dst, _NS)
    src32 = windows(src, pad_src, _NC * _NS)
    dst32 = windows(dst, pad_dst, _NC * _NS)

    ones_n = jnp.ones((n + 8,), jnp.float32)
    ones_w = jnp.ones((_W,), jnp.float32)
    deg = _sc_degree(dst16, ones_n, ones_w)
    deg2 = deg[:n].reshape(n, 1)

    def chunked(w, n_chunks, ch):
        return jnp.moveaxis(w.reshape(w.shape[0], n_chunks, ch), 1, 0)

    def aggregate(g, n_chunks, ch):
        zeros_rows = jnp.zeros((-(-(n // _NS) // 8) * 8, ch), jnp.float32)
        sr, dr = (src32, dst32) if n_chunks == 1 else (src16, dst16)
        agg = _sc_aggregate(sr, dr, g.reshape(n_chunks * n, ch), zeros_rows,
                            n=n, ch=ch, n_chunks=n_chunks)
        return agg.reshape(-1, n, ch)

    g1 = _tc_gmm(x, chunked(W1, 1, 128), deg=deg2)
    agg1 = aggregate(g1, 1, 128)
    g2 = _tc_gcn_fused(agg1, g1, deg2, b1.reshape(1, -1), chunked(W2, 2, 128))
    agg2 = aggregate(g2, 2, 128)
    g3 = _tc_gcn_fused(agg2, g2, deg2, b2.reshape(1, -1), chunked(W3, 4, 128))
    agg3 = aggregate(g3, 4, 128)
    pooled = _tc_pool_fused(agg3, g3, deg2, b3.reshape(1, -1),
                            batch.astype(jnp.int32).reshape(n, 1))
    return _tc_mlp(pooled, Wf1, bf1, Wf2, bf2)


# speedup vs baseline: 1.0127x; 1.0127x over previous
import functools

import jax
import jax.numpy as jnp
from jax import lax
import jax.experimental.pallas as pl
from jax.experimental.pallas import tpu as pltpu
from jax.experimental.pallas import tpu_sc as plsc

_NC = 2
_NS = 16
_W = 80
_G = 64
_RB = 1000


_NB = 4
_PW = 16


def _sc_degree(dst_r, ones_n, ones_w):
    np8 = ones_n.shape[0]
    nw = dst_r.shape[1]
    nblk = nw // _NB
    mesh = plsc.VectorSubcoreMesh(core_axis_name="c", subcore_axis_name="s")

    @functools.partial(
        pl.kernel,
        out_type=jax.ShapeDtypeStruct((np8,), jnp.float32),
        mesh=mesh,
        scratch_types=[
            pltpu.VMEM((nw, _W), jnp.int32),
            pltpu.VMEM((_W,), jnp.float32),
            pltpu.VMEM_SHARED((np8,), jnp.float32),
            [pltpu.SemaphoreType.DMA] * _NB,
        ],
    )
    def k(dst_h, ones_n_h, ones_w_h, out_h, dst2, onev, acc, sems):
        cid = lax.axis_index("c")
        sid = lax.axis_index("s")

        @pl.when(cid == 0)
        def _():
            @pl.when(sid == 0)
            def _():
                pltpu.sync_copy(ones_n_h, acc)
            pltpu.sync_copy(ones_w_h, onev)
            pltpu.sync_copy(dst_h.at[sid], dst2)
            plsc.subcore_barrier()

            def outer(k_):
                base = k_ * _NB
                for b in range(_NB):
                    @pl.when(k_ > 0)
                    def _():
                        pltpu.make_async_copy(
                            onev, acc.at[dst2.at[base - _NB + b]],
                            sems[b]).wait()
                    pltpu.async_copy(onev, acc.at[dst2.at[base + b]],
                                     sems[b], add=True)

            pl.loop(0, nblk)(outer)
            for b in range(_NB):
                pltpu.make_async_copy(
                    onev, acc.at[dst2.at[(nblk - 1) * _NB + b]],
                    sems[b]).wait()
            plsc.subcore_barrier()

            @pl.when(sid == 0)
            def _():
                pltpu.sync_copy(acc, out_h)

    return k(dst_r, ones_n, ones_w)


def _sc_aggregate(src, dst, g_flat, zeros_rows, *, n, ch, n_chunks):
    split = n_chunks == 1
    cpc = 1 if split else n_chunks // _NC
    nw = src.shape[1]
    n_out = _NC * n if split else n_chunks * n
    rpt_a = -(-(n // _NS) // 8) * 8
    rpt_b = n - (_NS - 1) * rpt_a
    mesh = plsc.VectorSubcoreMesh(core_axis_name="c", subcore_axis_name="s")

    npass = nw // _PW

    @functools.partial(
        pl.kernel,
        out_type=jax.ShapeDtypeStruct((n_out, ch), jnp.float32),
        mesh=mesh,
        scratch_types=[
            pltpu.VMEM((2 * _PW, _W), jnp.int32),
            pltpu.VMEM((2 * _PW, _W), jnp.int32),
            pltpu.VMEM((_NB, _W), jnp.int32),
            pltpu.VMEM((_NB * _W, ch), jnp.float32),
            pltpu.VMEM_SHARED((n + 8, ch), jnp.float32),
            [pltpu.SemaphoreType.DMA] * _NB,
            [pltpu.SemaphoreType.DMA] * 2,
        ],
    )
    def k(src_h, dst_h, g_h, z_h, out_h, src2, dst2, idxg_a, rows_a, acc,
          sems, psems):
        rows = [rows_a.at[pl.ds(b * _W, _W)] for b in range(_NB)]
        idxg = [idxg_a.at[b] for b in range(_NB)]
        cid = lax.axis_index("c")
        sid = lax.axis_index("s")
        slab = cid * _NS + sid if split else sid
        rbase = sid * rpt_a
        last = sid == _NS - 1

        for ci in range(cpc):
            chunk = cid * cpc + ci
            coff_in = 0 if split else chunk * n
            coff = cid * n if split else chunk * n

            def issue_gather(b, w):
                for j in range(_W // 16):
                    sl = pl.ds(j * 16, 16)
                    idxg[b][sl] = (src2[w, sl] if split
                                   else src2[w, sl] + coff_in)
                pltpu.async_copy(g_h.at[idxg[b]], rows[b], sems[b])

            @pl.when(jnp.logical_not(last))
            def _():
                pltpu.sync_copy(z_h, acc.at[pl.ds(rbase, rpt_a)])

            @pl.when(last)
            def _():
                pltpu.sync_copy(z_h.at[pl.ds(0, rpt_b)],
                                acc.at[pl.ds(rbase, rpt_b)])

            plsc.subcore_barrier()

            pltpu.sync_copy(src_h.at[slab, pl.ds(0, _PW)],
                            src2.at[pl.ds(0, _PW)])
            pltpu.sync_copy(dst_h.at[slab, pl.ds(0, _PW)],
                            dst2.at[pl.ds(0, _PW)])

            for p_ in range(npass):
                boff = (p_ % 2) * _PW
                noff = ((p_ + 1) % 2) * _PW
                if p_ + 1 < npass:
                    pltpu.async_copy(
                        src_h.at[slab, pl.ds((p_ + 1) * _PW, _PW)],
                        src2.at[pl.ds(noff, _PW)], psems[0])
                    pltpu.async_copy(
                        dst_h.at[slab, pl.ds((p_ + 1) * _PW, _PW)],
                        dst2.at[pl.ds(noff, _PW)], psems[1])

                for b in range(_NB):
                    issue_gather(b, boff + b)

                def outer(k_):
                    base = k_ * _NB
                    for b in range(_NB):
                        pltpu.make_async_copy(g_h.at[idxg[b]], rows[b],
                                              sems[b]).wait()
                        pltpu.async_copy(rows[b],
                                         acc.at[dst2.at[boff + base + b]],
                                         sems[b], add=True)
                    for b in range(_NB):
                        pltpu.make_async_copy(rows[b],
                                              acc.at[dst2.at[boff + base + b]],
                                              sems[b]).wait()

                        @pl.when(base + _NB + b < _PW)
                        def _():
                            issue_gather(b, boff + base + _NB + b)

                pl.loop(0, _PW // _NB)(outer)

                if p_ + 1 < npass:
                    pltpu.make_async_copy(
                        src_h.at[slab, pl.ds((p_ + 1) * _PW, _PW)],
                        src2.at[pl.ds(noff, _PW)], psems[0]).wait()
                    pltpu.make_async_copy(
                        dst_h.at[slab, pl.ds((p_ + 1) * _PW, _PW)],
                        dst2.at[pl.ds(noff, _PW)], psems[1]).wait()

            plsc.subcore_barrier()

            @pl.when(jnp.logical_not(last))
            def _():
                pltpu.sync_copy(acc.at[pl.ds(rbase, rpt_a)],
                                out_h.at[pl.ds(coff + rbase, rpt_a)])

            @pl.when(last)
            def _():
                pltpu.sync_copy(acc.at[pl.ds(rbase, rpt_b)],
                                out_h.at[pl.ds(coff + rbase, rpt_b)])

    return k(src, dst, g_flat, zeros_rows)


def _tc_gmm(p, wc, *, deg=None):
    n, din = p.shape
    n_chunks, _, ch = wc.shape

    def body_plain(p_ref, w_ref, o_ref):
        o_ref[0] = jnp.dot(p_ref[...], w_ref[0],
                           preferred_element_type=jnp.float32)

    def body_scaled(p_ref, w_ref, d_ref, o_ref):
        pv = p_ref[...] * lax.rsqrt(d_ref[...])
        o_ref[0] = jnp.dot(pv, w_ref[0], preferred_element_type=jnp.float32)

    in_specs = [
        pl.BlockSpec((_RB, din), lambda i, c: (i, 0)),
        pl.BlockSpec((1, din, ch), lambda i, c: (c, 0, 0)),
    ]
    args = [p, wc]
    body = body_plain
    if deg is not None:
        in_specs.append(pl.BlockSpec((_RB, 1), lambda i, c: (i, 0)))
        args.append(deg)
        body = body_scaled
    return pl.pallas_call(
        body,
        grid=(n // _RB, n_chunks),
        in_specs=in_specs,
        out_specs=pl.BlockSpec((1, _RB, ch), lambda i, c: (c, i, 0)),
        out_shape=jax.ShapeDtypeStruct((n_chunks, n, ch), jnp.float32),
    )(*args)


def _epi(a_ref, g_ref, d_ref, b_ref):
    dinv = lax.rsqrt(d_ref[...])
    a = a_ref[...]
    gg = g_ref[...]
    nc = gg.shape[0]
    if a.shape[0] == nc:
        parts = [a[i] + gg[i] for i in range(nc)]
    else:
        parts = [a[0] + a[1] + gg[0]]
    sc = parts[0] if len(parts) == 1 else jnp.concatenate(parts, axis=1)
    return jnp.maximum(dinv * sc + b_ref[...], 0.0), dinv


def _tc_gcn_fused(agg, g, deg, b, wc):
    nc_in, n, ch = g.shape
    m = agg.shape[0]
    nc_out, din, cho = wc.shape

    def body(a_ref, g_ref, d_ref, b_ref, w_ref, o_ref):
        h, dinv = _epi(a_ref, g_ref, d_ref, b_ref)
        p = h * dinv
        for c in range(nc_out):
            o_ref[c] = jnp.dot(p, w_ref[c], preferred_element_type=jnp.float32)

    return pl.pallas_call(
        body,
        grid=(n // _RB,),
        in_specs=[
            pl.BlockSpec((m, _RB, ch), lambda i: (0, i, 0)),
            pl.BlockSpec((nc_in, _RB, ch), lambda i: (0, i, 0)),
            pl.BlockSpec((_RB, 1), lambda i: (i, 0)),
            pl.BlockSpec((1, din), lambda i: (0, 0)),
            pl.BlockSpec((nc_out, din, cho), lambda i: (0, 0, 0)),
        ],
        out_specs=pl.BlockSpec((nc_out, _RB, cho), lambda i: (0, i, 0)),
        out_shape=jax.ShapeDtypeStruct((nc_out, n, cho), jnp.float32),
    )(agg, g, deg, b, wc)


def _tc_pool_fused(agg, g, deg, b, batch2):
    nc, n, ch = g.shape
    dout = nc * ch

    def body(a_ref, g_ref, d_ref, b_ref, bt_ref, o_ref):
        @pl.when(pl.program_id(0) == 0)
        def _():
            o_ref[...] = jnp.full((_G, dout), -jnp.inf, jnp.float32)

        hv, _ = _epi(a_ref, g_ref, d_ref, b_ref)
        bv = bt_ref[...]
        lo = bv[0, 0]
        hi = bv[_RB - 1, 0]

        def gbody(gi, _):
            mk = bv == gi
            part = jnp.max(jnp.where(mk, hv, -jnp.inf), axis=0, keepdims=True)
            cur = o_ref[pl.ds(gi, 1), :]
            o_ref[pl.ds(gi, 1), :] = jnp.maximum(cur, part)
            return ()

        lax.fori_loop(lo, hi + 1, gbody, ())

    return pl.pallas_call(
        body,
        grid=(n // _RB,),
        in_specs=[
            pl.BlockSpec((nc, _RB, ch), lambda i: (0, i, 0)),
            pl.BlockSpec((nc, _RB, ch), lambda i: (0, i, 0)),
            pl.BlockSpec((_RB, 1), lambda i: (i, 0)),
            pl.BlockSpec((1, dout), lambda i: (0, 0)),
            pl.BlockSpec((_RB, 1), lambda i: (i, 0)),
        ],
        out_specs=pl.BlockSpec((_G, dout), lambda i: (0, 0)),
        out_shape=jax.ShapeDtypeStruct((_G, dout), jnp.float32),
    )(agg, g, deg, b, batch2)


def _tc_mlp(pooled, wf1, bf1, wf2, bf2):
    def body(p_ref, w1_ref, b1_ref, w2_ref, b2_ref, o_ref):
        t = jnp.dot(p_ref[...], w1_ref[...], preferred_element_type=jnp.float32)
        t = jnp.maximum(t + b1_ref[...], 0.0)
        o_ref[...] = jnp.dot(t, w2_ref[...],
                             preferred_element_type=jnp.float32) + b2_ref[...]

    g, dout = pooled.shape[0], wf2.shape[1]
    return pl.pallas_call(
        body,
        out_shape=jax.ShapeDtypeStruct((g, dout), jnp.float32),
    )(pooled, wf1, bf1.reshape(1, -1), wf2, bf2.reshape(1, -1))


def kernel(x, edge_index, batch, W1, b1, W2, b2, W3, b3, Wf1, bf1, Wf2, bf2):
    n = x.shape[0]
    e = edge_index.shape[1]
    src = edge_index[0].astype(jnp.int32)
    dst = edge_index[1].astype(jnp.int32)

    ept16 = -(-(e // _NS) // (_W * _PW)) * (_W * _PW)
    padn = _NS * ept16 - e
    pad_src = (jnp.arange(padn, dtype=jnp.int32) % n)
    pad_dst = n + (jnp.arange(padn, dtype=jnp.int32) % 8)

    def windows(a, pad, groups):
        return jnp.concatenate(
            [a.reshape(groups, -1), pad.reshape(groups, -1)], axis=1
        ).reshape(groups, -1, _W)

    src16 = windows(src, pad_src, _NS)
    dst16 = windows(dst, pad_dst, _NS)
    src32 = windows(src, pad_src, _NC * _NS)
    dst32 = windows(dst, pad_dst, _NC * _NS)

    ones_n = jnp.ones((n + 8,), jnp.float32)
    ones_w = jnp.ones((_W,), jnp.float32)
    deg = _sc_degree(dst16, ones_n, ones_w)
    deg2 = deg[:n].reshape(n, 1)

    def chunked(w, n_chunks, ch):
        return jnp.moveaxis(w.reshape(w.shape[0], n_chunks, ch), 1, 0)

    def aggregate(g, n_chunks, ch):
        zeros_rows = jnp.zeros((-(-(n // _NS) // 8) * 8, ch), jnp.float32)
        sr, dr = (src32, dst32) if n_chunks == 1 else (src16, dst16)
        agg = _sc_aggregate(sr, dr, g.reshape(n_chunks * n, ch), zeros_rows,
                            n=n, ch=ch, n_chunks=n_chunks)
        return agg.reshape(-1, n, ch)

    g1 = _tc_gmm(x, chunked(W1, 1, 128), deg=deg2)
    agg1 = aggregate(g1, 1, 128)
    g2 = _tc_gcn_fused(agg1, g1, deg2, b1.reshape(1, -1), chunked(W2, 2, 128))
    agg2 = aggregate(g2, 2, 128)
    g3 = _tc_gcn_fused(agg2, g2, deg2, b2.reshape(1, -1), chunked(W3, 4, 128))
    agg3 = aggregate(g3, 4, 128)
    pooled = _tc_pool_fused(agg3, g3, deg2, b3.reshape(1, -1),
                            batch.astype(jnp.int32).reshape(n, 1))
    return _tc_mlp(pooled, Wf1, bf1, Wf2, bf2)
